# baseline (speedup 1.0000x reference)
import functools

import jax
import jax.numpy as jnp
from jax import lax
from jax.experimental import pallas as pl
from jax.experimental.pallas import tpu as pltpu
from jax.experimental.pallas import tpu_sc as plsc

N_BITS = 2
N_LEVELS = 2 ** N_BITS

NC = 2
NS = 16
NW = NC * NS
LANES = 16
CHUNK = 12544
N_ACC = 13

K_SC = 16
TC_BC = 32


def _moments(x, t1, t2, t3, av, tn, one, zero):
    c1 = jnp.where(x > t1, one, zero)
    c2 = jnp.where(x >= t2, one, zero)
    c3 = jnp.where(x > t3, one, zero)
    y1 = x * c1
    y2 = x * c2
    y3 = x * c3
    xx = x * x
    b = c1 + c2 + c3
    ad = jnp.abs(x - av * b)
    near = jnp.where(ad < tn, one, zero)
    return (c1, c2, c3, x, y1, y2, y3, xx, y1 * x, y2 * x, y3 * x, ad, near)


def _sc_vec_body(buf, i, acc, t1, t2, t3, av, tn):
    x = buf[pl.ds(i * LANES, LANES)]
    one = jnp.full((LANES,), 1.0, jnp.float32)
    zero = jnp.full((LANES,), 0.0, jnp.float32)
    terms = _moments(x, t1, t2, t3, av, tn, one, zero)
    return tuple(a + t for a, t in zip(acc, terms))


@functools.partial(jax.jit, static_argnames=("n",))
def _sc_reduce(acts_flat, alpha_vec, *, n):
    per_w = n // NW
    n_chunks = per_w // CHUNK
    assert per_w * NW == n and n_chunks * CHUNK == per_w and n_chunks % 2 == 0

    mesh = plsc.VectorSubcoreMesh(core_axis_name="c", subcore_axis_name="s",
                                  num_cores=NC, num_subcores=NS)

    @functools.partial(
        pl.kernel,
        out_type=jax.ShapeDtypeStruct((NW, N_ACC * LANES), jnp.float32),
        mesh=mesh,
        scratch_types=[
            pltpu.VMEM((CHUNK,), jnp.float32),
            pltpu.VMEM((CHUNK,), jnp.float32),
            pltpu.VMEM((LANES,), jnp.float32),
            pltpu.VMEM((N_ACC * LANES,), jnp.float32),
            pltpu.SemaphoreType.DMA,
            pltpu.SemaphoreType.DMA,
        ],
    )
    def sc_kernel(acts_hbm, av_hbm, out_hbm, buf0, buf1, pv, stage, sem0, sem1):
        wid = lax.axis_index("s") * NC + lax.axis_index("c")
        base = wid * per_w

        pltpu.sync_copy(av_hbm, pv)
        av = pv[...]
        t1 = av * 0.5
        t2 = av * 1.5
        t3 = av * 2.5
        tn = av * 0.01

        bufs = (buf0, buf1)
        sems = (sem0, sem1)

        def issue(c, p):
            pltpu.async_copy(acts_hbm.at[pl.ds(base + c * CHUNK, CHUNK)],
                             bufs[p], sems[p])

        def wait(p):
            pltpu.make_async_copy(acts_hbm.at[pl.ds(base, CHUNK)],
                                  bufs[p], sems[p]).wait()

        def process(p, acc):
            return lax.fori_loop(
                0, CHUNK // LANES,
                lambda i, a: _sc_vec_body(bufs[p], i, a, t1, t2, t3, av, tn),
                acc, unroll=4)

        issue(0, 0)

        acc0 = (jnp.zeros((LANES,), jnp.float32),) * N_ACC

        def pair_body(g, acc):
            c = 2 * g
            issue(c + 1, 1)
            wait(0)
            acc = process(0, acc)
            issue(c + 2, 0)
            wait(1)
            acc = process(1, acc)
            return acc

        acc = lax.fori_loop(0, n_chunks // 2 - 1, pair_body, acc0)
        issue(n_chunks - 1, 1)
        wait(0)
        acc = process(0, acc)
        wait(1)
        acc = process(1, acc)

        for r in range(N_ACC):
            stage[pl.ds(r * LANES, LANES)] = acc[r]
        pltpu.sync_copy(stage, out_hbm.at[wid])

    return sc_kernel(acts_flat, alpha_vec)


def _tc_body(params_ref, x_ref, out_ref, acc_ref):
    o = pl.program_id(0)
    c = pl.program_id(1)
    no = pl.num_programs(0)
    gc = pl.num_programs(1)

    @pl.when(jnp.logical_and(o == 0, c == 0))
    def _():
        acc_ref[...] = jnp.zeros_like(acc_ref)

    a = params_ref[0]
    one = jnp.float32(1.0)
    zero = jnp.float32(0.0)
    w = x_ref.shape[3]
    ones_col = jnp.ones((w, 1), jnp.float32)

    def face(i, accs):
        x = x_ref[0, i]
        terms = _moments(x, a * 0.5, a * 1.5, a * 2.5, a, a * 0.01, one, zero)
        return tuple(
            acc + jax.lax.dot(t, ones_col,
                              precision=jax.lax.Precision.DEFAULT)
            for acc, t in zip(accs, terms))

    acc0 = (jnp.zeros((x_ref.shape[2], 1), jnp.float32),) * N_ACC
    accs = lax.fori_loop(0, TC_BC, face, acc0, unroll=2)
    for j in range(N_ACC):
        acc_ref[j] += accs[j]

    @pl.when(jnp.logical_and(o == no - 1, c == gc - 1))
    def _():
        out_ref[...] = acc_ref[...]


@functools.partial(jax.jit, static_argnames=("k0", "n_out", "n_ch"))
def _tc_reduce(acts, params, *, k0, n_out, n_ch):
    gc = n_ch // TC_BC
    grid = (n_out, gc)
    h, w = acts.shape[2], acts.shape[3]
    return pl.pallas_call(
        _tc_body,
        grid=grid,
        in_specs=[
            pl.BlockSpec(memory_space=pltpu.SMEM),
            pl.BlockSpec((1, TC_BC, h, w), lambda o, c: (k0 + o, c, 0, 0)),
        ],
        out_specs=pl.BlockSpec((N_ACC, h, 1), lambda o, c: (0, 0, 0)),
        out_shape=jax.ShapeDtypeStruct((N_ACC, h, 1), jnp.float32),
        scratch_shapes=[pltpu.VMEM((N_ACC, h, 1), jnp.float32)],
        compiler_params=pltpu.CompilerParams(
            dimension_semantics=("arbitrary", "arbitrary")),
    )(params, acts)


def kernel(activations, alpha):
    dt = jnp.float32
    alpha = alpha.astype(dt)
    n = activations.size
    n_sc = K_SC * activations.shape[1] * activations.shape[2] * activations.shape[3]

    acts_sc = activations[:K_SC].reshape(-1)
    alpha_vec = jnp.full((LANES,), alpha, dt)
    sc_parts = _sc_reduce(acts_sc, alpha_vec, n=n_sc)

    params = jnp.full((8,), alpha, dt)
    tc_parts = _tc_reduce(activations, params, k0=K_SC,
                          n_out=activations.shape[0] - K_SC,
                          n_ch=activations.shape[1])

    tot = (sc_parts.reshape(NW, N_ACC, LANES).sum(axis=(0, 2))
           + tc_parts.sum(axis=(1, 2)))
    s1, s2, s3, S, P1, P2, P3, Q, R1, R2, R3, D, E = [tot[i] for i in range(N_ACC)]

    nf = jnp.asarray(n, dt)
    cnt = jnp.stack([nf - s1, s1 - s2, s2 - s3, s3])
    bsum = jnp.stack([S - P1, P1 - P2, P2 - P3, P3])
    bsq = jnp.stack([Q - R1, R1 - R2, R2 - R3, R3])

    levels = jnp.arange(N_LEVELS, dtype=dt) * alpha
    safe = jnp.maximum(cnt, 1.0)
    mean = bsum / safe
    mse = jnp.where(cnt > 0, (mean - levels) ** 2, 0.0)
    total_mse = jnp.sum(mse)
    var = (bsq - 2.0 * mean * bsum + cnt * mean * mean) / safe
    total_var = jnp.sum(jnp.where(cnt >= 2, var, 0.0))
    loss = total_mse + total_var

    qsq = bsq - 2.0 * levels * bsum + cnt * levels * levels
    quantization_mse = jnp.sum(qsq) / nf
    mean_distance = D / nf
    max_dist = alpha * 0.5
    effectiveness = jnp.clip(100.0 * (1.0 - mean_distance / (max_dist + 1e-12)),
                             0.0, 100.0)
    near_levels = (E / nf) * 100.0
    return (loss, total_mse, total_var, quantization_mse, mean_distance,
            effectiveness, near_levels)


# speedup vs baseline: 2.7962x; 2.7962x over previous
import functools

import jax
import jax.numpy as jnp
from jax import lax
from jax.experimental import pallas as pl
from jax.experimental.pallas import tpu as pltpu
from jax.experimental.pallas import tpu_sc as plsc

N_BITS = 2
N_LEVELS = 2 ** N_BITS

NC = 2
NS = 16
NW = NC * NS
LANES = 16
CHUNK = 12544
N_ACC = 13

K_SC = 8
TC_BC = 64


def _moments(x, t1, t2, t3, av, tn, one, zero):
    c1 = jnp.where(x > t1, one, zero)
    c2 = jnp.where(x >= t2, one, zero)
    c3 = jnp.where(x > t3, one, zero)
    y1 = x * c1
    y2 = x * c2
    y3 = x * c3
    xx = x * x
    b = c1 + c2 + c3
    ad = jnp.abs(x - av * b)
    near = jnp.where(ad < tn, one, zero)
    return (c1, c2, c3, x, y1, y2, y3, xx, y1 * x, y2 * x, y3 * x, ad, near)


def _sc_vec_body(buf, i, acc, t1, t2, t3, av, tn):
    x = buf[pl.ds(i * LANES, LANES)]
    one = jnp.full((LANES,), 1.0, jnp.float32)
    zero = jnp.full((LANES,), 0.0, jnp.float32)
    terms = _moments(x, t1, t2, t3, av, tn, one, zero)
    return tuple(a + t for a, t in zip(acc, terms))


@functools.partial(jax.jit, static_argnames=("n",))
def _sc_reduce(acts_flat, alpha_vec, *, n):
    per_w = n // NW
    n_chunks = per_w // CHUNK
    assert per_w * NW == n and n_chunks * CHUNK == per_w and n_chunks % 2 == 0

    mesh = plsc.VectorSubcoreMesh(core_axis_name="c", subcore_axis_name="s",
                                  num_cores=NC, num_subcores=NS)

    @functools.partial(
        pl.kernel,
        out_type=jax.ShapeDtypeStruct((NW, N_ACC * LANES), jnp.float32),
        mesh=mesh,
        scratch_types=[
            pltpu.VMEM((CHUNK,), jnp.float32),
            pltpu.VMEM((CHUNK,), jnp.float32),
            pltpu.VMEM((LANES,), jnp.float32),
            pltpu.VMEM((N_ACC * LANES,), jnp.float32),
            pltpu.SemaphoreType.DMA,
            pltpu.SemaphoreType.DMA,
        ],
    )
    def sc_kernel(acts_hbm, av_hbm, out_hbm, buf0, buf1, pv, stage, sem0, sem1):
        wid = lax.axis_index("s") * NC + lax.axis_index("c")
        base = wid * per_w

        pltpu.sync_copy(av_hbm, pv)
        av = pv[...]
        t1 = av * 0.5
        t2 = av * 1.5
        t3 = av * 2.5
        tn = av * 0.01

        bufs = (buf0, buf1)
        sems = (sem0, sem1)

        def issue(c, p):
            pltpu.async_copy(acts_hbm.at[pl.ds(base + c * CHUNK, CHUNK)],
                             bufs[p], sems[p])

        def wait(p):
            pltpu.make_async_copy(acts_hbm.at[pl.ds(base, CHUNK)],
                                  bufs[p], sems[p]).wait()

        def process(p, acc):
            return lax.fori_loop(
                0, CHUNK // LANES,
                lambda i, a: _sc_vec_body(bufs[p], i, a, t1, t2, t3, av, tn),
                acc, unroll=4)

        issue(0, 0)

        acc0 = (jnp.zeros((LANES,), jnp.float32),) * N_ACC

        def pair_body(g, acc):
            c = 2 * g
            issue(c + 1, 1)
            wait(0)
            acc = process(0, acc)
            issue(c + 2, 0)
            wait(1)
            acc = process(1, acc)
            return acc

        acc = lax.fori_loop(0, n_chunks // 2 - 1, pair_body, acc0)
        issue(n_chunks - 1, 1)
        wait(0)
        acc = process(0, acc)
        wait(1)
        acc = process(1, acc)

        for r in range(N_ACC):
            stage[pl.ds(r * LANES, LANES)] = acc[r]
        pltpu.sync_copy(stage, out_hbm.at[wid])

    return sc_kernel(acts_flat, alpha_vec)


def _tc_body(params_ref, x_ref, out_ref, acc_ref):
    o = pl.program_id(0)
    c = pl.program_id(1)
    no = pl.num_programs(0)
    gc = pl.num_programs(1)

    @pl.when(jnp.logical_and(o == 0, c == 0))
    def _():
        acc_ref[...] = jnp.zeros_like(acc_ref)

    a = params_ref[0]
    one = jnp.float32(1.0)
    zero = jnp.float32(0.0)

    def _fold56(t):
        r = t[0:8]
        for k in range(1, 7):
            r = r + t[8 * k:8 * (k + 1)]
        return r

    def face(i, accs):
        x = x_ref[0, i]
        terms = _moments(x, a * 0.5, a * 1.5, a * 2.5, a, a * 0.01, one, zero)
        return tuple(acc + _fold56(t) for acc, t in zip(accs, terms))

    acc0 = (jnp.zeros((8, x_ref.shape[3]), jnp.float32),) * N_ACC
    accs = lax.fori_loop(0, TC_BC, face, acc0, unroll=2)
    for j in range(N_ACC):
        acc_ref[j] += accs[j]

    @pl.when(jnp.logical_and(o == no - 1, c == gc - 1))
    def _():
        out_ref[...] = acc_ref[...]


@functools.partial(jax.jit, static_argnames=("k0", "n_out", "n_ch"))
def _tc_reduce(acts, params, *, k0, n_out, n_ch):
    gc = n_ch // TC_BC
    grid = (n_out, gc)
    h, w = acts.shape[2], acts.shape[3]
    return pl.pallas_call(
        _tc_body,
        grid=grid,
        in_specs=[
            pl.BlockSpec(memory_space=pltpu.SMEM),
            pl.BlockSpec((1, TC_BC, h, w), lambda o, c: (k0 + o, c, 0, 0)),
        ],
        out_specs=pl.BlockSpec((N_ACC, 8, w), lambda o, c: (0, 0, 0)),
        out_shape=jax.ShapeDtypeStruct((N_ACC, 8, w), jnp.float32),
        scratch_shapes=[pltpu.VMEM((N_ACC, 8, w), jnp.float32)],
        compiler_params=pltpu.CompilerParams(
            dimension_semantics=("arbitrary", "arbitrary")),
    )(params, acts)


def kernel(activations, alpha):
    dt = jnp.float32
    alpha = alpha.astype(dt)
    n = activations.size
    n_sc = K_SC * activations.shape[1] * activations.shape[2] * activations.shape[3]

    acts_sc = activations[:K_SC].reshape(-1)
    alpha_vec = jnp.full((LANES,), alpha, dt)
    sc_parts = _sc_reduce(acts_sc, alpha_vec, n=n_sc)

    params = jnp.full((8,), alpha, dt)
    tc_parts = _tc_reduce(activations, params, k0=K_SC,
                          n_out=activations.shape[0] - K_SC,
                          n_ch=activations.shape[1])

    tot = (sc_parts.reshape(NW, N_ACC, LANES).sum(axis=(0, 2))
           + tc_parts.sum(axis=(1, 2)))
    s1, s2, s3, S, P1, P2, P3, Q, R1, R2, R3, D, E = [tot[i] for i in range(N_ACC)]

    nf = jnp.asarray(n, dt)
    cnt = jnp.stack([nf - s1, s1 - s2, s2 - s3, s3])
    bsum = jnp.stack([S - P1, P1 - P2, P2 - P3, P3])
    bsq = jnp.stack([Q - R1, R1 - R2, R2 - R3, R3])

    levels = jnp.arange(N_LEVELS, dtype=dt) * alpha
    safe = jnp.maximum(cnt, 1.0)
    mean = bsum / safe
    mse = jnp.where(cnt > 0, (mean - levels) ** 2, 0.0)
    total_mse = jnp.sum(mse)
    var = (bsq - 2.0 * mean * bsum + cnt * mean * mean) / safe
    total_var = jnp.sum(jnp.where(cnt >= 2, var, 0.0))
    loss = total_mse + total_var

    qsq = bsq - 2.0 * levels * bsum + cnt * levels * levels
    quantization_mse = jnp.sum(qsq) / nf
    mean_distance = D / nf
    max_dist = alpha * 0.5
    effectiveness = jnp.clip(100.0 * (1.0 - mean_distance / (max_dist + 1e-12)),
                             0.0, 100.0)
    near_levels = (E / nf) * 100.0
    return (loss, total_mse, total_var, quantization_mse, mean_distance,
            effectiveness, near_levels)


# speedup vs baseline: 2.8789x; 1.0296x over previous
import functools

import jax
import jax.numpy as jnp
from jax import lax
from jax.experimental import pallas as pl
from jax.experimental.pallas import tpu as pltpu
from jax.experimental.pallas import tpu_sc as plsc

N_BITS = 2
N_LEVELS = 2 ** N_BITS

NC = 2
NS = 16
NW = NC * NS
LANES = 16
CHUNK = 12544
N_ACC = 13

K_SC = 4
TC_BC = 64


def _moments(x, t1, t2, t3, av, tn, one, zero):
    c1 = jnp.where(x > t1, one, zero)
    c2 = jnp.where(x >= t2, one, zero)
    c3 = jnp.where(x > t3, one, zero)
    y1 = x * c1
    y2 = x * c2
    y3 = x * c3
    xx = x * x
    b = c1 + c2 + c3
    ad = jnp.abs(x - av * b)
    near = jnp.where(ad < tn, one, zero)
    return (c1, c2, c3, x, y1, y2, y3, xx, y1 * x, y2 * x, y3 * x, ad, near)


def _sc_vec_body(buf, i, acc, t1, t2, t3, av, tn):
    x = buf[pl.ds(i * LANES, LANES)]
    one = jnp.full((LANES,), 1.0, jnp.float32)
    zero = jnp.full((LANES,), 0.0, jnp.float32)
    terms = _moments(x, t1, t2, t3, av, tn, one, zero)
    return tuple(a + t for a, t in zip(acc, terms))


@functools.partial(jax.jit, static_argnames=("n",))
def _sc_reduce(acts_flat, alpha_vec, *, n):
    per_w = n // NW
    n_chunks = per_w // CHUNK
    assert per_w * NW == n and n_chunks * CHUNK == per_w and n_chunks % 2 == 0

    mesh = plsc.VectorSubcoreMesh(core_axis_name="c", subcore_axis_name="s",
                                  num_cores=NC, num_subcores=NS)

    @functools.partial(
        pl.kernel,
        out_type=jax.ShapeDtypeStruct((NW, N_ACC * LANES), jnp.float32),
        mesh=mesh,
        scratch_types=[
            pltpu.VMEM((CHUNK,), jnp.float32),
            pltpu.VMEM((CHUNK,), jnp.float32),
            pltpu.VMEM((LANES,), jnp.float32),
            pltpu.VMEM((N_ACC * LANES,), jnp.float32),
            pltpu.SemaphoreType.DMA,
            pltpu.SemaphoreType.DMA,
        ],
    )
    def sc_kernel(acts_hbm, av_hbm, out_hbm, buf0, buf1, pv, stage, sem0, sem1):
        wid = lax.axis_index("s") * NC + lax.axis_index("c")
        base = wid * per_w

        pltpu.sync_copy(av_hbm, pv)
        av = pv[...]
        t1 = av * 0.5
        t2 = av * 1.5
        t3 = av * 2.5
        tn = av * 0.01

        bufs = (buf0, buf1)
        sems = (sem0, sem1)

        def issue(c, p):
            pltpu.async_copy(acts_hbm.at[pl.ds(base + c * CHUNK, CHUNK)],
                             bufs[p], sems[p])

        def wait(p):
            pltpu.make_async_copy(acts_hbm.at[pl.ds(base, CHUNK)],
                                  bufs[p], sems[p]).wait()

        def process(p, acc):
            return lax.fori_loop(
                0, CHUNK // LANES,
                lambda i, a: _sc_vec_body(bufs[p], i, a, t1, t2, t3, av, tn),
                acc, unroll=4)

        issue(0, 0)

        acc0 = (jnp.zeros((LANES,), jnp.float32),) * N_ACC

        def pair_body(g, acc):
            c = 2 * g
            issue(c + 1, 1)
            wait(0)
            acc = process(0, acc)
            issue(c + 2, 0)
            wait(1)
            acc = process(1, acc)
            return acc

        acc = lax.fori_loop(0, n_chunks // 2 - 1, pair_body, acc0)
        issue(n_chunks - 1, 1)
        wait(0)
        acc = process(0, acc)
        wait(1)
        acc = process(1, acc)

        for r in range(N_ACC):
            stage[pl.ds(r * LANES, LANES)] = acc[r]
        pltpu.sync_copy(stage, out_hbm.at[wid])

    return sc_kernel(acts_flat, alpha_vec)


def _tc_body(params_ref, x_ref, out_ref, acc_ref):
    o = pl.program_id(0)
    c = pl.program_id(1)
    no = pl.num_programs(0)
    gc = pl.num_programs(1)

    @pl.when(jnp.logical_and(o == 0, c == 0))
    def _():
        acc_ref[...] = jnp.zeros_like(acc_ref)

    a = params_ref[0]
    one = jnp.float32(1.0)
    zero = jnp.float32(0.0)

    def _fold56(t):
        r = t[0:8]
        for k in range(1, 7):
            r = r + t[8 * k:8 * (k + 1)]
        return r

    def face(i, accs):
        x = x_ref[0, i]
        terms = _moments(x, a * 0.5, a * 1.5, a * 2.5, a, a * 0.01, one, zero)
        return tuple(acc + _fold56(t) for acc, t in zip(accs, terms))

    acc0 = (jnp.zeros((8, x_ref.shape[3]), jnp.float32),) * N_ACC
    accs = lax.fori_loop(0, TC_BC, face, acc0, unroll=4)
    for j in range(N_ACC):
        acc_ref[j] += accs[j]

    @pl.when(jnp.logical_and(o == no - 1, c == gc - 1))
    def _():
        out_ref[...] = acc_ref[...]


@functools.partial(jax.jit, static_argnames=("k0", "n_out", "n_ch"))
def _tc_reduce(acts, params, *, k0, n_out, n_ch):
    gc = n_ch // TC_BC
    grid = (n_out, gc)
    h, w = acts.shape[2], acts.shape[3]
    return pl.pallas_call(
        _tc_body,
        grid=grid,
        in_specs=[
            pl.BlockSpec(memory_space=pltpu.SMEM),
            pl.BlockSpec((1, TC_BC, h, w), lambda o, c: (k0 + o, c, 0, 0)),
        ],
        out_specs=pl.BlockSpec((N_ACC, 8, w), lambda o, c: (0, 0, 0)),
        out_shape=jax.ShapeDtypeStruct((N_ACC, 8, w), jnp.float32),
        scratch_shapes=[pltpu.VMEM((N_ACC, 8, w), jnp.float32)],
        compiler_params=pltpu.CompilerParams(
            dimension_semantics=("arbitrary", "arbitrary")),
    )(params, acts)


def kernel(activations, alpha):
    dt = jnp.float32
    alpha = alpha.astype(dt)
    n = activations.size
    n_sc = K_SC * activations.shape[1] * activations.shape[2] * activations.shape[3]

    acts_sc = activations[:K_SC].reshape(-1)
    alpha_vec = jnp.full((LANES,), alpha, dt)
    sc_parts = _sc_reduce(acts_sc, alpha_vec, n=n_sc)

    params = jnp.full((8,), alpha, dt)
    tc_parts = _tc_reduce(activations, params, k0=K_SC,
                          n_out=activations.shape[0] - K_SC,
                          n_ch=activations.shape[1])

    tot = (sc_parts.reshape(NW, N_ACC, LANES).sum(axis=(0, 2))
           + tc_parts.sum(axis=(1, 2)))
    s1, s2, s3, S, P1, P2, P3, Q, R1, R2, R3, D, E = [tot[i] for i in range(N_ACC)]

    nf = jnp.asarray(n, dt)
    cnt = jnp.stack([nf - s1, s1 - s2, s2 - s3, s3])
    bsum = jnp.stack([S - P1, P1 - P2, P2 - P3, P3])
    bsq = jnp.stack([Q - R1, R1 - R2, R2 - R3, R3])

    levels = jnp.arange(N_LEVELS, dtype=dt) * alpha
    safe = jnp.maximum(cnt, 1.0)
    mean = bsum / safe
    mse = jnp.where(cnt > 0, (mean - levels) ** 2, 0.0)
    total_mse = jnp.sum(mse)
    var = (bsq - 2.0 * mean * bsum + cnt * mean * mean) / safe
    total_var = jnp.sum(jnp.where(cnt >= 2, var, 0.0))
    loss = total_mse + total_var

    qsq = bsq - 2.0 * levels * bsum + cnt * levels * levels
    quantization_mse = jnp.sum(qsq) / nf
    mean_distance = D / nf
    max_dist = alpha * 0.5
    effectiveness = jnp.clip(100.0 * (1.0 - mean_distance / (max_dist + 1e-12)),
                             0.0, 100.0)
    near_levels = (E / nf) * 100.0
    return (loss, total_mse, total_var, quantization_mse, mean_distance,
            effectiveness, near_levels)
